# Initial kernel scaffold; baseline (speedup 1.0000x reference)
import jax
import jax.numpy as jnp
from jax.experimental import pallas as pl


def kernel(x, table):
    raise NotImplementedError("write your pallas kernel here")


# speedup vs baseline: 1.0243x; 1.0243x over previous
import functools

import jax
import jax.numpy as jnp
from jax import lax
from jax.experimental import pallas as pl
from jax.experimental.pallas import tpu as pltpu
from jax.experimental.pallas import tpu_sc as plsc

_N_VOCAB = 1000
_D = 1000
_BATCH = 1024
_SEQ = 50
_NC = 2
_NS = 16
_NW = _NC * _NS
_B_TOT = _BATCH * _SEQ
_B_PER_W = _B_TOT // _NW
_C = 40
_NCHUNK = _B_PER_W // _C

_mesh = plsc.VectorSubcoreMesh(core_axis_name="c", subcore_axis_name="s")


@functools.partial(
    pl.kernel,
    mesh=_mesh,
    out_type=jax.ShapeDtypeStruct((_B_TOT, _D), jnp.float32),
    compiler_params=pltpu.CompilerParams(use_tc_tiling_on_sc=False),
    scratch_types=[
        pltpu.VMEM((_C,), jnp.int32),
        pltpu.VMEM((_C,), jnp.int32),
        pltpu.VMEM((_C, _D), jnp.float32),
        pltpu.VMEM((_C, _D), jnp.float32),
        pltpu.SemaphoreType.DMA,
        pltpu.SemaphoreType.DMA,
        pltpu.SemaphoreType.DMA,
        pltpu.SemaphoreType.DMA,
    ],
)
def _sc_gather(x_hbm, table_hbm, out_hbm,
               idx0, idx1, buf0, buf1, sg0, sg1, ss0, ss1):
    wid = lax.axis_index("s") * _NC + lax.axis_index("c")
    base = wid * _B_PER_W

    def body(c2, carry):
        o0 = base + (2 * c2) * _C
        o1 = o0 + _C
        pltpu.sync_copy(x_hbm.at[pl.ds(o0, _C)], idx0)
        g0 = pltpu.async_copy(table_hbm.at[idx0], buf0, sg0)
        pltpu.sync_copy(x_hbm.at[pl.ds(o1, _C)], idx1)
        g1 = pltpu.async_copy(table_hbm.at[idx1], buf1, sg1)
        g0.wait()
        s0 = pltpu.async_copy(buf0, out_hbm.at[pl.ds(o0, _C)], ss0)
        g1.wait()
        s1 = pltpu.async_copy(buf1, out_hbm.at[pl.ds(o1, _C)], ss1)
        s0.wait()
        s1.wait()
        return carry

    lax.fori_loop(0, _NCHUNK // 2, body, 0)


def kernel(x, table):
    out = _sc_gather(x.reshape(_B_TOT), table)
    return out.reshape(_BATCH, _SEQ, _D)
